# Initial kernel scaffold; baseline (speedup 1.0000x reference)
import jax
import jax.numpy as jnp
from jax.experimental import pallas as pl


def kernel(note_id, phoneme_id, slur, phone_progress, note_table, phoneme_table, slur_table, pp_table, W1, b1, W2, b2):
    raise NotImplementedError("write your pallas kernel here")


# speedup vs baseline: 5.3372x; 5.3372x over previous
import functools

import jax
import jax.numpy as jnp
from jax import lax
from jax.experimental import pallas as pl
from jax.experimental.pallas import tpu as pltpu
from jax.experimental.pallas import tpu_sc as plsc

NC = 2
NS = 16
NW = NC * NS

G = 128
NBUF = 2


@functools.lru_cache(maxsize=None)
def _sc_gather(n_tok, note_d, phon_d, ng):
    mesh = plsc.VectorSubcoreMesh(core_axis_name="c", subcore_axis_name="s")
    per_w = ng * G

    @functools.partial(
        pl.kernel,
        out_type=[
            jax.ShapeDtypeStruct((n_tok, note_d), jnp.float32),
            jax.ShapeDtypeStruct((n_tok, phon_d), jnp.float32),
        ],
        mesh=mesh,
        scratch_types=[
            pltpu.VMEM((ng, G), jnp.int32),
            pltpu.VMEM((ng, G), jnp.int32),
            pltpu.VMEM((NBUF, G, note_d), jnp.float32),
            pltpu.VMEM((NBUF, G, phon_d), jnp.float32),
            pltpu.SemaphoreType.DMA((NBUF,)),
        ],
    )
    def gather_kernel(idx_n_hbm, idx_p_hbm, note_hbm, phon_hbm,
                      out_n, out_p, idxn, idxp, rn, rp, sems):
        wid = lax.axis_index("s") * NC + lax.axis_index("c")
        base = wid * per_w

        pltpu.sync_copy(idx_n_hbm.at[wid], idxn)
        pltpu.sync_copy(idx_p_hbm.at[wid], idxp)

        def start(g, b):
            pltpu.async_copy(note_hbm.at[idxn.at[g]], rn.at[b], sems.at[b])
            pltpu.async_copy(phon_hbm.at[idxp.at[g]], rp.at[b], sems.at[b])

        def drain(b):
            pltpu.make_async_copy(out_n.at[pl.ds(0, G)], rn.at[b], sems.at[b]).wait()
            pltpu.make_async_copy(out_p.at[pl.ds(0, G)], rp.at[b], sems.at[b]).wait()

        def flush(g, b):
            tok = pl.multiple_of(base + g * G, G)
            pltpu.sync_copy(rn.at[b], out_n.at[pl.ds(tok, G)])
            pltpu.sync_copy(rp.at[b], out_p.at[pl.ds(tok, G)])

        for b in range(NBUF):
            start(b, b)

        def body(i, _):
            g0 = i * NBUF
            for b in range(NBUF):
                g = g0 + b
                drain(b)
                flush(g, b)

                @pl.when(g + NBUF < ng)
                def _():
                    start(g + NBUF, b)
            return 0

        lax.fori_loop(0, ng // NBUF, body, 0)

    return gather_kernel


@functools.lru_cache(maxsize=None)
def _tc_mlp(n_tok, note_d, phon_d, slur_v, slur_d, pp_v, pp_d, cond, blk):
    grid = (n_tok // blk,)

    def mlp_body(n, p, s_id, pp_id, s_tab, pp_tab,
                 w1n, w1p, w1s, w1pp, b1, w2, b2, out):
        h = jnp.dot(n[...], w1n[...], preferred_element_type=jnp.float32)
        h = h + jnp.dot(p[...], w1p[...], preferred_element_type=jnp.float32)
        sw = jnp.dot(s_tab[...], w1s[...], preferred_element_type=jnp.float32)
        pw = jnp.dot(pp_tab[...], w1pp[...], preferred_element_type=jnp.float32)
        oh_s = (s_id[...] == lax.broadcasted_iota(jnp.int32, (1, slur_v), 1)
                ).astype(jnp.float32)
        oh_pp = (pp_id[...] == lax.broadcasted_iota(jnp.int32, (1, pp_v), 1)
                 ).astype(jnp.float32)
        h = h + jnp.dot(oh_s, sw, preferred_element_type=jnp.float32)
        h = h + jnp.dot(oh_pp, pw, preferred_element_type=jnp.float32)
        h = h + b1[...]
        h = h * jax.nn.sigmoid(h)
        out[...] = jnp.dot(h, w2[...], preferred_element_type=jnp.float32) + b2[...]

    def row_spec(d):
        return pl.BlockSpec((blk, d), lambda i: (i, 0))

    def full_spec(r, c):
        return pl.BlockSpec((r, c), lambda i: (0, 0))

    return pl.pallas_call(
        mlp_body,
        grid=grid,
        in_specs=[
            row_spec(note_d), row_spec(phon_d), row_spec(1), row_spec(1),
            full_spec(slur_v, slur_d), full_spec(pp_v, pp_d),
            full_spec(note_d, cond), full_spec(phon_d, cond),
            full_spec(slur_d, cond), full_spec(pp_d, cond),
            full_spec(1, cond), full_spec(cond, cond), full_spec(1, cond),
        ],
        out_specs=pl.BlockSpec((blk, cond), lambda i: (i, 0)),
        out_shape=jax.ShapeDtypeStruct((n_tok, cond), jnp.float32),
    )


def kernel(note_id, phoneme_id, slur, phone_progress, note_table,
           phoneme_table, slur_table, pp_table, W1, b1, W2, b2):
    B, L = note_id.shape
    n_tok = B * L
    note_d = note_table.shape[1]
    phon_d = phoneme_table.shape[1]
    slur_v, slur_d = slur_table.shape
    pp_v, pp_d = pp_table.shape
    cond = W2.shape[1]
    per_w = n_tok // NW
    ng = per_w // G

    idx_n = note_id.astype(jnp.int32).reshape(NW, ng, G)
    idx_p = phoneme_id.astype(jnp.int32).reshape(NW, ng, G)

    rows_n, rows_p = _sc_gather(n_tok, note_d, phon_d, ng)(
        idx_n, idx_p, note_table, phoneme_table)

    w1n = W1[:note_d]
    w1p = W1[note_d:note_d + phon_d]
    w1s = W1[note_d + phon_d:note_d + phon_d + slur_d]
    w1pp = W1[note_d + phon_d + slur_d:]

    s_id = jnp.clip(slur, 0, slur_v - 1).astype(jnp.int32).reshape(n_tok, 1)
    pp_id = phone_progress.astype(jnp.int32).reshape(n_tok, 1)

    out = _tc_mlp(n_tok, note_d, phon_d, slur_v, slur_d, pp_v, pp_d, cond, 1024)(
        rows_n, rows_p, s_id, pp_id, slur_table, pp_table,
        w1n, w1p, w1s, w1pp, b1.reshape(1, cond), W2, b2.reshape(1, cond))

    return out.reshape(B, L, cond)


# speedup vs baseline: 5.3587x; 1.0040x over previous
import functools

import jax
import jax.numpy as jnp
from jax import lax
from jax.experimental import pallas as pl
from jax.experimental.pallas import tpu as pltpu
from jax.experimental.pallas import tpu_sc as plsc

NC = 2
NS = 16
NW = NC * NS

G = 128
NBUF = 2


@functools.lru_cache(maxsize=None)
def _sc_gather(n_tok, note_d, phon_d, ng):
    mesh = plsc.VectorSubcoreMesh(core_axis_name="c", subcore_axis_name="s")
    per_w = ng * G

    @functools.partial(
        pl.kernel,
        out_type=[
            jax.ShapeDtypeStruct((n_tok, note_d), jnp.float32),
            jax.ShapeDtypeStruct((n_tok, phon_d), jnp.float32),
        ],
        mesh=mesh,
        scratch_types=[
            pltpu.VMEM((ng, G), jnp.int32),
            pltpu.VMEM((ng, G), jnp.int32),
            pltpu.VMEM((NBUF, G, note_d), jnp.float32),
            pltpu.VMEM((NBUF, G, phon_d), jnp.float32),
            pltpu.SemaphoreType.DMA((NBUF,)),
        ],
    )
    def gather_kernel(idx_n_hbm, idx_p_hbm, note_hbm, phon_hbm,
                      out_n, out_p, idxn, idxp, rn, rp, sems):
        wid = lax.axis_index("s") * NC + lax.axis_index("c")
        base = wid * per_w

        pltpu.sync_copy(idx_n_hbm.at[wid], idxn)
        pltpu.sync_copy(idx_p_hbm.at[wid], idxp)

        def start(g, b):
            pltpu.async_copy(note_hbm.at[idxn.at[g]], rn.at[b], sems.at[b])
            pltpu.async_copy(phon_hbm.at[idxp.at[g]], rp.at[b], sems.at[b])

        def drain(b):
            pltpu.make_async_copy(out_n.at[pl.ds(0, G)], rn.at[b], sems.at[b]).wait()
            pltpu.make_async_copy(out_p.at[pl.ds(0, G)], rp.at[b], sems.at[b]).wait()

        def flush(g, b):
            tok = pl.multiple_of(base + g * G, G)
            pltpu.sync_copy(rn.at[b], out_n.at[pl.ds(tok, G)])
            pltpu.sync_copy(rp.at[b], out_p.at[pl.ds(tok, G)])

        for b in range(NBUF):
            start(b, b)

        def body(i, _):
            g0 = i * NBUF
            for b in range(NBUF):
                g = g0 + b
                drain(b)
                flush(g, b)

                @pl.when(g + NBUF < ng)
                def _():
                    start(g + NBUF, b)
            return 0

        lax.fori_loop(0, ng // NBUF, body, 0)

    return gather_kernel


@functools.lru_cache(maxsize=None)
def _tc_mlp(n_tok, note_d, phon_d, slur_v, slur_d, pp_v, pp_d, cond, blk):
    grid = (n_tok // blk,)

    bf16 = jnp.bfloat16

    def mlp_body(n, p, s_id, pp_id, s_tab, pp_tab,
                 w1n, w1p, w1s, w1pp, b1, w2, b2, out):
        h = jnp.dot(n[...].astype(bf16), w1n[...].astype(bf16),
                    preferred_element_type=jnp.float32)
        h = h + jnp.dot(p[...].astype(bf16), w1p[...].astype(bf16),
                        preferred_element_type=jnp.float32)
        sw = jnp.dot(s_tab[...], w1s[...], preferred_element_type=jnp.float32)
        pw = jnp.dot(pp_tab[...], w1pp[...], preferred_element_type=jnp.float32)
        oh_s = (s_id[...] == lax.broadcasted_iota(jnp.int32, (1, slur_v), 1)
                ).astype(bf16)
        oh_pp = (pp_id[...] == lax.broadcasted_iota(jnp.int32, (1, pp_v), 1)
                 ).astype(bf16)
        h = h + jnp.dot(oh_s, sw.astype(bf16), preferred_element_type=jnp.float32)
        h = h + jnp.dot(oh_pp, pw.astype(bf16), preferred_element_type=jnp.float32)
        h = h + b1[...]
        h = h * jax.nn.sigmoid(h)
        out[...] = jnp.dot(h.astype(bf16), w2[...].astype(bf16),
                           preferred_element_type=jnp.float32) + b2[...]

    def row_spec(d):
        return pl.BlockSpec((blk, d), lambda i: (i, 0))

    def full_spec(r, c):
        return pl.BlockSpec((r, c), lambda i: (0, 0))

    return pl.pallas_call(
        mlp_body,
        grid=grid,
        in_specs=[
            row_spec(note_d), row_spec(phon_d), row_spec(1), row_spec(1),
            full_spec(slur_v, slur_d), full_spec(pp_v, pp_d),
            full_spec(note_d, cond), full_spec(phon_d, cond),
            full_spec(slur_d, cond), full_spec(pp_d, cond),
            full_spec(1, cond), full_spec(cond, cond), full_spec(1, cond),
        ],
        out_specs=pl.BlockSpec((blk, cond), lambda i: (i, 0)),
        out_shape=jax.ShapeDtypeStruct((n_tok, cond), jnp.float32),
    )


def kernel(note_id, phoneme_id, slur, phone_progress, note_table,
           phoneme_table, slur_table, pp_table, W1, b1, W2, b2):
    B, L = note_id.shape
    n_tok = B * L
    note_d = note_table.shape[1]
    phon_d = phoneme_table.shape[1]
    slur_v, slur_d = slur_table.shape
    pp_v, pp_d = pp_table.shape
    cond = W2.shape[1]
    per_w = n_tok // NW
    ng = per_w // G

    idx_n = note_id.astype(jnp.int32).reshape(NW, ng, G)
    idx_p = phoneme_id.astype(jnp.int32).reshape(NW, ng, G)

    rows_n, rows_p = _sc_gather(n_tok, note_d, phon_d, ng)(
        idx_n, idx_p, note_table, phoneme_table)

    w1n = W1[:note_d]
    w1p = W1[note_d:note_d + phon_d]
    w1s = W1[note_d + phon_d:note_d + phon_d + slur_d]
    w1pp = W1[note_d + phon_d + slur_d:]

    s_id = jnp.clip(slur, 0, slur_v - 1).astype(jnp.int32).reshape(n_tok, 1)
    pp_id = phone_progress.astype(jnp.int32).reshape(n_tok, 1)

    out = _tc_mlp(n_tok, note_d, phon_d, slur_v, slur_d, pp_v, pp_d, cond, 1024)(
        rows_n, rows_p, s_id, pp_id, slur_table, pp_table,
        w1n, w1p, w1s, w1pp, b1.reshape(1, cond), W2, b2.reshape(1, cond))

    return out.reshape(B, L, cond)


# speedup vs baseline: 6.5887x; 1.2295x over previous
import functools

import jax
import jax.numpy as jnp
from jax import lax
from jax.experimental import pallas as pl
from jax.experimental.pallas import tpu as pltpu
from jax.experimental.pallas import tpu_sc as plsc

NC = 2
NS = 16
NW = NC * NS

G = 128
NBUF = 2


@functools.lru_cache(maxsize=None)
def _sc_gather(n_tok, note_d, phon_d, ng):
    mesh = plsc.VectorSubcoreMesh(core_axis_name="c", subcore_axis_name="s")
    per_w = ng * G

    @functools.partial(
        pl.kernel,
        out_type=[
            jax.ShapeDtypeStruct((n_tok, note_d), jnp.float32),
            jax.ShapeDtypeStruct((n_tok, phon_d), jnp.float32),
        ],
        mesh=mesh,
        scratch_types=[
            pltpu.VMEM((ng, G), jnp.int32),
            pltpu.VMEM((ng, G), jnp.int32),
            pltpu.VMEM((NBUF, G, note_d), jnp.float32),
            pltpu.VMEM((NBUF, G, phon_d), jnp.float32),
            pltpu.SemaphoreType.DMA((NBUF,)),
        ],
    )
    def gather_kernel(idx_n_hbm, idx_p_hbm, note_hbm, phon_hbm,
                      out_n, out_p, idxn, idxp, rn, rp, sems):
        wid = lax.axis_index("s") * NC + lax.axis_index("c")
        base = wid * per_w

        pltpu.sync_copy(idx_n_hbm.at[wid], idxn)
        pltpu.sync_copy(idx_p_hbm.at[wid], idxp)

        def start(g, b):
            pltpu.async_copy(note_hbm.at[idxn.at[g]], rn.at[b], sems.at[b])
            pltpu.async_copy(phon_hbm.at[idxp.at[g]], rp.at[b], sems.at[b])

        def drain(b):
            pltpu.make_async_copy(out_n.at[pl.ds(0, G)], rn.at[b], sems.at[b]).wait()
            pltpu.make_async_copy(out_p.at[pl.ds(0, G)], rp.at[b], sems.at[b]).wait()

        def flush(g, b):
            tok = pl.multiple_of(base + g * G, G)
            pltpu.sync_copy(rn.at[b], out_n.at[pl.ds(tok, G)])
            pltpu.sync_copy(rp.at[b], out_p.at[pl.ds(tok, G)])

        for b in range(NBUF):
            start(b, b)

        def body(i, _):
            g0 = i * NBUF
            for b in range(NBUF):
                g = g0 + b
                drain(b)
                flush(g, b)

                @pl.when(g + NBUF < ng)
                def _():
                    start(g + NBUF, b)
            return 0

        lax.fori_loop(0, ng // NBUF, body, 0)

    return gather_kernel


@functools.lru_cache(maxsize=None)
def _tc_mlp(n_tok, note_d, phon_d, slur_v, slur_d, pp_v, pp_d, cond, blk):
    grid = (n_tok // blk,)
    nsub = blk // 128
    combo = slur_v * pp_v

    bf16 = jnp.bfloat16

    def mlp_body(n, p, c_id, s_tab, pp_tab,
                 w1n, w1p, w1s, w1pp, b1, w2, b2, out):
        h = jnp.dot(n[...].astype(bf16), w1n[...].astype(bf16),
                    preferred_element_type=jnp.float32)
        h = h + jnp.dot(p[...].astype(bf16), w1p[...].astype(bf16),
                        preferred_element_type=jnp.float32)
        sw = jnp.dot(s_tab[...], w1s[...], preferred_element_type=jnp.float32)
        pw = jnp.dot(pp_tab[...], w1pp[...], preferred_element_type=jnp.float32)
        spw = (jnp.repeat(sw, pp_v, axis=0) + jnp.tile(pw, (slur_v, 1))
               ).astype(bf16)
        cid = c_id[0]
        iota_c = lax.broadcasted_iota(jnp.int32, (combo, 128), 0)
        parts = []
        for r in range(nsub):
            ohT = (cid[r:r + 1, :] == iota_c).astype(bf16)
            parts.append(lax.dot_general(
                ohT, spw, (((0,), (0,)), ((), ())),
                preferred_element_type=jnp.float32))
        h = h + jnp.concatenate(parts, axis=0)
        h = h + b1[...]
        h = h * jax.nn.sigmoid(h)
        out[...] = jnp.dot(h.astype(bf16), w2[...].astype(bf16),
                           preferred_element_type=jnp.float32) + b2[...]

    def row_spec(d):
        return pl.BlockSpec((blk, d), lambda i: (i, 0))

    def full_spec(r, c):
        return pl.BlockSpec((r, c), lambda i: (0, 0))

    return pl.pallas_call(
        mlp_body,
        grid=grid,
        in_specs=[
            row_spec(note_d), row_spec(phon_d),
            pl.BlockSpec((1, nsub, 128), lambda i: (i, 0, 0)),
            full_spec(slur_v, slur_d), full_spec(pp_v, pp_d),
            full_spec(note_d, cond), full_spec(phon_d, cond),
            full_spec(slur_d, cond), full_spec(pp_d, cond),
            full_spec(1, cond), full_spec(cond, cond), full_spec(1, cond),
        ],
        out_specs=pl.BlockSpec((blk, cond), lambda i: (i, 0)),
        out_shape=jax.ShapeDtypeStruct((n_tok, cond), jnp.float32),
    )


def kernel(note_id, phoneme_id, slur, phone_progress, note_table,
           phoneme_table, slur_table, pp_table, W1, b1, W2, b2):
    B, L = note_id.shape
    n_tok = B * L
    note_d = note_table.shape[1]
    phon_d = phoneme_table.shape[1]
    slur_v, slur_d = slur_table.shape
    pp_v, pp_d = pp_table.shape
    cond = W2.shape[1]
    per_w = n_tok // NW
    ng = per_w // G

    idx_n = note_id.astype(jnp.int32).reshape(NW, ng, G)
    idx_p = phoneme_id.astype(jnp.int32).reshape(NW, ng, G)

    rows_n, rows_p = _sc_gather(n_tok, note_d, phon_d, ng)(
        idx_n, idx_p, note_table, phoneme_table)

    w1n = W1[:note_d]
    w1p = W1[note_d:note_d + phon_d]
    w1s = W1[note_d + phon_d:note_d + phon_d + slur_d]
    w1pp = W1[note_d + phon_d + slur_d:]

    blk = 1024
    c_id = (jnp.clip(slur, 0, slur_v - 1).astype(jnp.int32) * pp_v
            + phone_progress.astype(jnp.int32)).reshape(n_tok // blk, blk // 128, 128)

    out = _tc_mlp(n_tok, note_d, phon_d, slur_v, slur_d, pp_v, pp_d, cond, blk)(
        rows_n, rows_p, c_id, slur_table, pp_table,
        w1n, w1p, w1s, w1pp, b1.reshape(1, cond), W2, b2.reshape(1, cond))

    return out.reshape(B, L, cond)


# speedup vs baseline: 7.2885x; 1.1062x over previous
import functools

import jax
import jax.numpy as jnp
from jax import lax
from jax.experimental import pallas as pl
from jax.experimental.pallas import tpu as pltpu
from jax.experimental.pallas import tpu_sc as plsc

NC = 2
NS = 16
NW = NC * NS

G = 128
NBUF = 2


@functools.lru_cache(maxsize=None)
def _sc_gather(n_tok, note_d, phon_d, ng):
    mesh = plsc.VectorSubcoreMesh(core_axis_name="c", subcore_axis_name="s")
    per_w = ng * G

    @functools.partial(
        pl.kernel,
        out_type=[
            jax.ShapeDtypeStruct((n_tok, note_d), jnp.float32),
            jax.ShapeDtypeStruct((n_tok, phon_d), jnp.float32),
        ],
        mesh=mesh,
        scratch_types=[
            pltpu.VMEM((ng, G), jnp.int32),
            pltpu.VMEM((ng, G), jnp.int32),
            pltpu.VMEM((NBUF, G, note_d), jnp.float32),
            pltpu.VMEM((NBUF, G, phon_d), jnp.float32),
            pltpu.SemaphoreType.DMA((NBUF,)),
        ],
    )
    def gather_kernel(idx_n_hbm, idx_p_hbm, note_hbm, phon_hbm,
                      out_n, out_p, idxn, idxp, rn, rp, sems):
        wid = lax.axis_index("s") * NC + lax.axis_index("c")
        base = wid * per_w

        pltpu.sync_copy(idx_n_hbm.at[wid], idxn)
        pltpu.sync_copy(idx_p_hbm.at[wid], idxp)

        def start(g, b):
            pltpu.async_copy(note_hbm.at[idxn.at[g]], rn.at[b], sems.at[b])
            pltpu.async_copy(phon_hbm.at[idxp.at[g]], rp.at[b], sems.at[b])

        def drain(b):
            pltpu.make_async_copy(out_n.at[pl.ds(0, G)], rn.at[b], sems.at[b]).wait()
            pltpu.make_async_copy(out_p.at[pl.ds(0, G)], rp.at[b], sems.at[b]).wait()

        def flush(g, b):
            tok = pl.multiple_of(base + g * G, G)
            pltpu.sync_copy(rn.at[b], out_n.at[pl.ds(tok, G)])
            pltpu.sync_copy(rp.at[b], out_p.at[pl.ds(tok, G)])

        for b in range(NBUF):
            start(b, b)

        def body(i, _):
            g0 = i * NBUF
            for b in range(NBUF):
                g = g0 + b
                drain(b)
                flush(g, b)

                @pl.when(g + NBUF < ng)
                def _():
                    start(g + NBUF, b)
            return 0

        lax.fori_loop(0, ng // NBUF, body, 0)

    return gather_kernel


@functools.lru_cache(maxsize=None)
def _tc_mlp(n_tok, note_d, phon_d, slur_v, slur_d, pp_v, pp_d, cond, blk,
            chunk_blks, base_blk, out_tok):
    grid = (chunk_blks,)
    nsub = blk // 128
    combo = slur_v * pp_v

    bf16 = jnp.bfloat16

    def mlp_body(*refs):
        if base_blk > 0:
            refs = refs[1:]
        n, p, c_id, s_tab, pp_tab, w1n, w1p, w1s, w1pp, b1, w2, b2, out = refs
        h = jnp.dot(n[...].astype(bf16), w1n[...].astype(bf16),
                    preferred_element_type=jnp.float32)
        h = h + jnp.dot(p[...].astype(bf16), w1p[...].astype(bf16),
                        preferred_element_type=jnp.float32)
        sw = jnp.dot(s_tab[...], w1s[...], preferred_element_type=jnp.float32)
        pw = jnp.dot(pp_tab[...], w1pp[...], preferred_element_type=jnp.float32)
        spw = (jnp.repeat(sw, pp_v, axis=0) + jnp.tile(pw, (slur_v, 1))
               ).astype(bf16)
        cid = c_id[0]
        iota_c = lax.broadcasted_iota(jnp.int32, (combo, 128), 0)
        parts = []
        for r in range(nsub):
            ohT = (cid[r:r + 1, :] == iota_c).astype(bf16)
            parts.append(lax.dot_general(
                ohT, spw, (((0,), (0,)), ((), ())),
                preferred_element_type=jnp.float32))
        h = h + jnp.concatenate(parts, axis=0)
        h = h + b1[...]
        h = h * jax.nn.sigmoid(h)
        out[...] = jnp.dot(h.astype(bf16), w2[...].astype(bf16),
                           preferred_element_type=jnp.float32) + b2[...]

    def row_spec(d):
        return pl.BlockSpec((blk, d), lambda i: (i, 0))

    def full_spec(r, c):
        return pl.BlockSpec((r, c), lambda i: (0, 0))

    in_specs = [
        row_spec(note_d), row_spec(phon_d),
        pl.BlockSpec((1, nsub, 128), lambda i: (i, 0, 0)),
        full_spec(slur_v, slur_d), full_spec(pp_v, pp_d),
        full_spec(note_d, cond), full_spec(phon_d, cond),
        full_spec(slur_d, cond), full_spec(pp_d, cond),
        full_spec(1, cond), full_spec(cond, cond), full_spec(1, cond),
    ]
    aliases = {}
    if base_blk > 0:
        in_specs = [pl.BlockSpec(memory_space=pl.MemorySpace.ANY)] + in_specs
        aliases = {0: 0}
    return pl.pallas_call(
        mlp_body,
        grid=grid,
        in_specs=in_specs,
        out_specs=pl.BlockSpec((blk, cond), lambda i: (base_blk + i, 0)),
        out_shape=jax.ShapeDtypeStruct((out_tok, cond), jnp.float32),
        input_output_aliases=aliases,
    )


def kernel(note_id, phoneme_id, slur, phone_progress, note_table,
           phoneme_table, slur_table, pp_table, W1, b1, W2, b2):
    B, L = note_id.shape
    n_tok = B * L
    note_d = note_table.shape[1]
    phon_d = phoneme_table.shape[1]
    slur_v, slur_d = slur_table.shape
    pp_v, pp_d = pp_table.shape
    cond = W2.shape[1]
    blk = 1024

    nchunks = 1
    for k in (5, 4, 2):
        ctok = n_tok // k
        if (n_tok % k == 0 and ctok % (NW * G) == 0 and ctok % blk == 0
                and (ctok // (NW * G)) % NBUF == 0):
            nchunks = k
            break
    ctok = n_tok // nchunks
    ng = ctok // (NW * G)

    idx_n = note_id.astype(jnp.int32).reshape(nchunks, NW, ng, G)
    idx_p = phoneme_id.astype(jnp.int32).reshape(nchunks, NW, ng, G)

    w1n = W1[:note_d]
    w1p = W1[note_d:note_d + phon_d]
    w1s = W1[note_d + phon_d:note_d + phon_d + slur_d]
    w1pp = W1[note_d + phon_d + slur_d:]
    b1r = b1.reshape(1, cond)
    b2r = b2.reshape(1, cond)

    c_id = (jnp.clip(slur, 0, slur_v - 1).astype(jnp.int32) * pp_v
            + phone_progress.astype(jnp.int32)
            ).reshape(nchunks, ctok // blk, blk // 128, 128)

    sc = _sc_gather(ctok, note_d, phon_d, ng)
    rows = [sc(idx_n[k], idx_p[k], note_table, phoneme_table)
            for k in range(nchunks)]

    out = None
    cblks = ctok // blk
    for k in range(nchunks):
        mlp = _tc_mlp(ctok, note_d, phon_d, slur_v, slur_d, pp_v, pp_d,
                      cond, blk, cblks, k * cblks, n_tok)
        args = (rows[k][0], rows[k][1], c_id[k], slur_table, pp_table,
                w1n, w1p, w1s, w1pp, b1r, W2, b2r)
        out = mlp(*args) if k == 0 else mlp(out, *args)

    return out.reshape(B, L, cond)


# speedup vs baseline: 8.6180x; 1.1824x over previous
import functools

import jax
import jax.numpy as jnp
from jax import lax
from jax.experimental import pallas as pl
from jax.experimental.pallas import tpu as pltpu
from jax.experimental.pallas import tpu_sc as plsc

NC = 2
NS = 16
NW = NC * NS

G = 128
NBUF = 2


@functools.lru_cache(maxsize=None)
def _sc_gather(n_tok, note_d, phon_d, ng, dtype):
    mesh = plsc.VectorSubcoreMesh(core_axis_name="c", subcore_axis_name="s")
    per_w = ng * G

    @functools.partial(
        pl.kernel,
        out_type=[
            jax.ShapeDtypeStruct((n_tok, note_d), dtype),
            jax.ShapeDtypeStruct((n_tok, phon_d), dtype),
        ],
        mesh=mesh,
        scratch_types=[
            pltpu.VMEM((ng, G), jnp.int32),
            pltpu.VMEM((ng, G), jnp.int32),
            pltpu.VMEM((NBUF, G, note_d), dtype),
            pltpu.VMEM((NBUF, G, phon_d), dtype),
            pltpu.SemaphoreType.DMA((NBUF,)),
        ],
    )
    def gather_kernel(idx_n_hbm, idx_p_hbm, note_hbm, phon_hbm,
                      out_n, out_p, idxn, idxp, rn, rp, sems):
        wid = lax.axis_index("s") * NC + lax.axis_index("c")
        base = wid * per_w

        pltpu.sync_copy(idx_n_hbm.at[wid], idxn)
        pltpu.sync_copy(idx_p_hbm.at[wid], idxp)

        def start(g, b):
            pltpu.async_copy(note_hbm.at[idxn.at[g]], rn.at[b], sems.at[b])
            pltpu.async_copy(phon_hbm.at[idxp.at[g]], rp.at[b], sems.at[b])

        def drain(b):
            pltpu.make_async_copy(out_n.at[pl.ds(0, G)], rn.at[b], sems.at[b]).wait()
            pltpu.make_async_copy(out_p.at[pl.ds(0, G)], rp.at[b], sems.at[b]).wait()

        def flush(g, b):
            tok = pl.multiple_of(base + g * G, G)
            pltpu.sync_copy(rn.at[b], out_n.at[pl.ds(tok, G)])
            pltpu.sync_copy(rp.at[b], out_p.at[pl.ds(tok, G)])

        for b in range(NBUF):
            start(b, b)

        def body(i, _):
            g0 = i * NBUF
            for b in range(NBUF):
                g = g0 + b
                drain(b)
                flush(g, b)

                @pl.when(g + NBUF < ng)
                def _():
                    start(g + NBUF, b)
            return 0

        lax.fori_loop(0, ng // NBUF, body, 0)

    return gather_kernel


@functools.lru_cache(maxsize=None)
def _tc_mlp(n_tok, note_d, phon_d, slur_v, slur_d, pp_v, pp_d, cond, blk,
            chunk_blks, base_blk, out_tok):
    grid = (chunk_blks,)
    nsub = blk // 128
    combo = slur_v * pp_v

    bf16 = jnp.bfloat16

    def mlp_body(*refs):
        if base_blk > 0:
            refs = refs[1:]
        n, p, c_id, s_tab, pp_tab, w1n, w1p, w1s, w1pp, b1, w2, b2, out = refs
        h = jnp.dot(n[...].astype(bf16), w1n[...].astype(bf16),
                    preferred_element_type=jnp.float32)
        h = h + jnp.dot(p[...].astype(bf16), w1p[...].astype(bf16),
                        preferred_element_type=jnp.float32)
        sw = jnp.dot(s_tab[...], w1s[...], preferred_element_type=jnp.float32)
        pw = jnp.dot(pp_tab[...], w1pp[...], preferred_element_type=jnp.float32)
        spw = (jnp.repeat(sw, pp_v, axis=0) + jnp.tile(pw, (slur_v, 1))
               ).astype(bf16)
        cid = c_id[0]
        iota_c = lax.broadcasted_iota(jnp.int32, (combo, 128), 0)
        parts = []
        for r in range(nsub):
            ohT = (cid[r:r + 1, :] == iota_c).astype(bf16)
            parts.append(lax.dot_general(
                ohT, spw, (((0,), (0,)), ((), ())),
                preferred_element_type=jnp.float32))
        h = h + jnp.concatenate(parts, axis=0)
        h = h + b1[...]
        h = h * jax.nn.sigmoid(h)
        out[...] = jnp.dot(h.astype(bf16), w2[...].astype(bf16),
                           preferred_element_type=jnp.float32) + b2[...]

    def row_spec(d):
        return pl.BlockSpec((blk, d), lambda i: (i, 0))

    def full_spec(r, c):
        return pl.BlockSpec((r, c), lambda i: (0, 0))

    in_specs = [
        row_spec(note_d), row_spec(phon_d),
        pl.BlockSpec((1, nsub, 128), lambda i: (i, 0, 0)),
        full_spec(slur_v, slur_d), full_spec(pp_v, pp_d),
        full_spec(note_d, cond), full_spec(phon_d, cond),
        full_spec(slur_d, cond), full_spec(pp_d, cond),
        full_spec(1, cond), full_spec(cond, cond), full_spec(1, cond),
    ]
    aliases = {}
    if base_blk > 0:
        in_specs = [pl.BlockSpec(memory_space=pl.MemorySpace.ANY)] + in_specs
        aliases = {0: 0}
    return pl.pallas_call(
        mlp_body,
        grid=grid,
        in_specs=in_specs,
        out_specs=pl.BlockSpec((blk, cond), lambda i: (base_blk + i, 0)),
        out_shape=jax.ShapeDtypeStruct((out_tok, cond), jnp.float32),
        input_output_aliases=aliases,
    )


def kernel(note_id, phoneme_id, slur, phone_progress, note_table,
           phoneme_table, slur_table, pp_table, W1, b1, W2, b2):
    B, L = note_id.shape
    n_tok = B * L
    note_d = note_table.shape[1]
    phon_d = phoneme_table.shape[1]
    slur_v, slur_d = slur_table.shape
    pp_v, pp_d = pp_table.shape
    cond = W2.shape[1]
    blk = 2048

    nchunks = 1
    for k in (5, 4, 2):
        ctok = n_tok // k
        if (n_tok % k == 0 and ctok % (NW * G) == 0 and ctok % blk == 0
                and (ctok // (NW * G)) % NBUF == 0):
            nchunks = k
            break
    ctok = n_tok // nchunks
    ng = ctok // (NW * G)

    idx_n = note_id.astype(jnp.int32).reshape(nchunks, NW, ng, G)
    idx_p = phoneme_id.astype(jnp.int32).reshape(nchunks, NW, ng, G)

    w1n = W1[:note_d]
    w1p = W1[note_d:note_d + phon_d]
    w1s = W1[note_d + phon_d:note_d + phon_d + slur_d]
    w1pp = W1[note_d + phon_d + slur_d:]
    b1r = b1.reshape(1, cond)
    b2r = b2.reshape(1, cond)

    c_id = (jnp.clip(slur, 0, slur_v - 1).astype(jnp.int32) * pp_v
            + phone_progress.astype(jnp.int32)
            ).reshape(nchunks, ctok // blk, blk // 128, 128)

    sc = _sc_gather(ctok, note_d, phon_d, ng, jnp.float32)
    rows = [sc(idx_n[k], idx_p[k], note_table, phoneme_table)
            for k in range(nchunks)]

    out = None
    cblks = ctok // blk
    for k in range(nchunks):
        mlp = _tc_mlp(ctok, note_d, phon_d, slur_v, slur_d, pp_v, pp_d,
                      cond, blk, cblks, k * cblks, n_tok)
        args = (rows[k][0], rows[k][1], c_id[k], slur_table, pp_table,
                w1n, w1p, w1s, w1pp, b1r, W2, b2r)
        out = mlp(*args) if k == 0 else mlp(out, *args)

    return out.reshape(B, L, cond)


# speedup vs baseline: 9.1780x; 1.0650x over previous
import functools

import jax
import jax.numpy as jnp
from jax import lax
from jax.experimental import pallas as pl
from jax.experimental.pallas import tpu as pltpu
from jax.experimental.pallas import tpu_sc as plsc

NC = 2
NS = 16
NW = NC * NS

G = 128
NBUF = 2


@functools.lru_cache(maxsize=None)
def _sc_gather(n_tok, note_d, phon_d, ng, dtype):
    mesh = plsc.VectorSubcoreMesh(core_axis_name="c", subcore_axis_name="s")
    per_w = ng * G

    @functools.partial(
        pl.kernel,
        out_type=[
            jax.ShapeDtypeStruct((n_tok, note_d), dtype),
            jax.ShapeDtypeStruct((n_tok, phon_d), dtype),
        ],
        mesh=mesh,
        scratch_types=[
            pltpu.VMEM((ng, G), jnp.int32),
            pltpu.VMEM((ng, G), jnp.int32),
            pltpu.VMEM((NBUF, G, note_d), dtype),
            pltpu.VMEM((NBUF, G, phon_d), dtype),
            pltpu.SemaphoreType.DMA((NBUF,)),
        ],
    )
    def gather_kernel(idx_n_hbm, idx_p_hbm, note_hbm, phon_hbm,
                      out_n, out_p, idxn, idxp, rn, rp, sems):
        wid = lax.axis_index("s") * NC + lax.axis_index("c")
        base = wid * per_w

        pltpu.sync_copy(idx_n_hbm.at[wid], idxn)
        pltpu.sync_copy(idx_p_hbm.at[wid], idxp)

        def start(g, b):
            pltpu.async_copy(note_hbm.at[idxn.at[g]], rn.at[b], sems.at[b])
            pltpu.async_copy(phon_hbm.at[idxp.at[g]], rp.at[b], sems.at[b])

        def drain(b):
            pltpu.make_async_copy(out_n.at[pl.ds(0, G)], rn.at[b], sems.at[b]).wait()
            pltpu.make_async_copy(out_p.at[pl.ds(0, G)], rp.at[b], sems.at[b]).wait()

        def flush(g, b):
            tok = pl.multiple_of(base + g * G, G)
            pltpu.sync_copy(rn.at[b], out_n.at[pl.ds(tok, G)])
            pltpu.sync_copy(rp.at[b], out_p.at[pl.ds(tok, G)])

        for b in range(NBUF):
            start(b, b)

        def body(i, _):
            g0 = i * NBUF
            for b in range(NBUF):
                g = g0 + b
                drain(b)
                flush(g, b)

                @pl.when(g + NBUF < ng)
                def _():
                    start(g + NBUF, b)
            return 0

        lax.fori_loop(0, ng // NBUF, body, 0)

    return gather_kernel


@functools.lru_cache(maxsize=None)
def _tc_mlp(n_tok, note_d, phon_d, slur_v, slur_d, pp_v, pp_d, cond, blk,
            chunk_blks, base_blk, out_tok):
    grid = (chunk_blks,)
    nsub = blk // 128
    combo = slur_v * pp_v

    bf16 = jnp.bfloat16

    def mlp_body(*refs):
        if base_blk > 0:
            refs = refs[1:]
        n, p, c_id, s_tab, pp_tab, w1n, w1p, w1s, w1pp, b1, w2, b2, out = refs
        h = jnp.dot(n[...].astype(bf16), w1n[...].astype(bf16),
                    preferred_element_type=jnp.float32)
        h = h + jnp.dot(p[...].astype(bf16), w1p[...].astype(bf16),
                        preferred_element_type=jnp.float32)
        sw = jnp.dot(s_tab[...], w1s[...], preferred_element_type=jnp.float32)
        pw = jnp.dot(pp_tab[...], w1pp[...], preferred_element_type=jnp.float32)
        spw = (jnp.repeat(sw, pp_v, axis=0) + jnp.tile(pw, (slur_v, 1))
               ).astype(bf16)
        cid = c_id[0]
        iota_c = lax.broadcasted_iota(jnp.int32, (combo, 128), 0)
        parts = []
        for r in range(nsub):
            ohT = (cid[r:r + 1, :] == iota_c).astype(bf16)
            parts.append(lax.dot_general(
                ohT, spw, (((0,), (0,)), ((), ())),
                preferred_element_type=jnp.float32))
        h = h + jnp.concatenate(parts, axis=0)
        h = h + b1[...]
        h = h * jax.nn.sigmoid(h)
        out[...] = jnp.dot(h.astype(bf16), w2[...].astype(bf16),
                           preferred_element_type=jnp.float32) + b2[...]

    def row_spec(d):
        return pl.BlockSpec((blk, d), lambda i: (i, 0))

    def full_spec(r, c):
        return pl.BlockSpec((r, c), lambda i: (0, 0))

    in_specs = [
        row_spec(note_d), row_spec(phon_d),
        pl.BlockSpec((1, nsub, 128), lambda i: (i, 0, 0)),
        full_spec(slur_v, slur_d), full_spec(pp_v, pp_d),
        full_spec(note_d, cond), full_spec(phon_d, cond),
        full_spec(slur_d, cond), full_spec(pp_d, cond),
        full_spec(1, cond), full_spec(cond, cond), full_spec(1, cond),
    ]
    aliases = {}
    if base_blk > 0:
        in_specs = [pl.BlockSpec(memory_space=pl.MemorySpace.ANY)] + in_specs
        aliases = {0: 0}
    return pl.pallas_call(
        mlp_body,
        grid=grid,
        in_specs=in_specs,
        out_specs=pl.BlockSpec((blk, cond), lambda i: (base_blk + i, 0)),
        out_shape=jax.ShapeDtypeStruct((out_tok, cond), jnp.float32),
        input_output_aliases=aliases,
    )


def kernel(note_id, phoneme_id, slur, phone_progress, note_table,
           phoneme_table, slur_table, pp_table, W1, b1, W2, b2):
    B, L = note_id.shape
    n_tok = B * L
    note_d = note_table.shape[1]
    phon_d = phoneme_table.shape[1]
    slur_v, slur_d = slur_table.shape
    pp_v, pp_d = pp_table.shape
    cond = W2.shape[1]
    blk = 4096

    nchunks = 1
    for k in (5, 4, 2):
        ctok = n_tok // k
        if (n_tok % k == 0 and ctok % (NW * G) == 0 and ctok % blk == 0
                and (ctok // (NW * G)) % NBUF == 0):
            nchunks = k
            break
    ctok = n_tok // nchunks
    ng = ctok // (NW * G)

    idx_n = note_id.astype(jnp.int32).reshape(nchunks, NW, ng, G)
    idx_p = phoneme_id.astype(jnp.int32).reshape(nchunks, NW, ng, G)

    w1n = W1[:note_d]
    w1p = W1[note_d:note_d + phon_d]
    w1s = W1[note_d + phon_d:note_d + phon_d + slur_d]
    w1pp = W1[note_d + phon_d + slur_d:]
    b1r = b1.reshape(1, cond)
    b2r = b2.reshape(1, cond)

    c_id = (jnp.clip(slur, 0, slur_v - 1).astype(jnp.int32) * pp_v
            + phone_progress.astype(jnp.int32)
            ).reshape(nchunks, ctok // blk, blk // 128, 128)

    sc = _sc_gather(ctok, note_d, phon_d, ng, jnp.float32)
    rows = [sc(idx_n[k], idx_p[k], note_table, phoneme_table)
            for k in range(nchunks)]

    out = None
    cblks = ctok // blk
    for k in range(nchunks):
        mlp = _tc_mlp(ctok, note_d, phon_d, slur_v, slur_d, pp_v, pp_d,
                      cond, blk, cblks, k * cblks, n_tok)
        args = (rows[k][0], rows[k][1], c_id[k], slur_table, pp_table,
                w1n, w1p, w1s, w1pp, b1r, W2, b2r)
        out = mlp(*args) if k == 0 else mlp(out, *args)

    return out.reshape(B, L, cond)
